# baseline (speedup 1.0000x reference)
import functools

import jax
import jax.numpy as jnp
from jax import lax
from jax.experimental import pallas as pl
from jax.experimental.pallas import tpu as pltpu
from jax.experimental.pallas import tpu_sc as plsc

G = 128
N = 200000
L = 16
W = 400
GPC = W // L
NCH = N // W
NC = 2
NS = 16
NW = NC * NS
KMAX = (NCH + NW - 1) // NW
RB = 16

_MESH = plsc.VectorSubcoreMesh(core_axis_name="c", subcore_axis_name="s")
_PARAMS = pltpu.CompilerParams(use_tc_tiling_on_sc=False,
                               needs_layout_passes=False)


def _widx():
    return lax.axis_index("s") * NC + lax.axis_index("c")


@functools.partial(
    pl.kernel,
    out_type=[
        jax.ShapeDtypeStruct((N,), jnp.float32),
        jax.ShapeDtypeStruct((N,), jnp.int32),
        jax.ShapeDtypeStruct((NW, G, L), jnp.float32),
    ],
    mesh=_MESH,
    compiler_params=_PARAMS,
    scratch_types=[
        pltpu.VMEM((G, W), jnp.float32),
        pltpu.VMEM((G, W), jnp.float32),
        pltpu.VMEM((G, L), jnp.float32),
        pltpu.VMEM((W,), jnp.float32),
        pltpu.VMEM((W,), jnp.int32),
        pltpu.VMEM((W,), jnp.int32),
        pltpu.SemaphoreType.DMA,
        pltpu.SemaphoreType.DMA,
    ],
)
def _k1(ov_hbm, maxov_hbm, pre_hbm, part_hbm,
        buf0, buf1, racc, cmbuf, aibuf, prebuf, sem0, sem1):
    w = _widx()

    def init_racc(r, _):
        racc[r] = jnp.full((L,), -1.0, jnp.float32)
        return 0
    lax.fori_loop(0, G, init_racc, 0)

    def chunk_of(k):
        return k * NW + w

    def start(k, buf, sem):
        c = chunk_of(k)

        @pl.when(c < NCH)
        def _():
            pltpu.make_async_copy(
                ov_hbm.at[:, pl.ds(c * W, W)], buf, sem).start()

    def compute(k, buf, sem):
        c = chunk_of(k)

        @pl.when(c < NCH)
        def _():
            pltpu.make_async_copy(
                ov_hbm.at[:, pl.ds(c * W, W)], buf, sem).wait()

            def init_g(g, _):
                cmbuf[pl.ds(g * L, L)] = jnp.full((L,), -1.0, jnp.float32)
                aibuf[pl.ds(g * L, L)] = jnp.zeros((L,), jnp.int32)
                return 0
            lax.fori_loop(0, GPC, init_g, 0)

            def rb_body(rb, _):
                r0 = rb * RB

                raccs = [racc[r0 + i] for i in range(RB)]
                for g in range(GPC):
                    gl = g * L
                    cm = cmbuf[pl.ds(gl, L)]
                    ai = aibuf[pl.ds(gl, L)]
                    for i in range(RB):
                        v = buf[r0 + i, pl.ds(gl, L)]
                        m = v > cm
                        cm = jnp.where(m, v, cm)
                        ai = jnp.where(
                            m, jnp.zeros((L,), jnp.int32) + (r0 + i), ai)
                        raccs[i] = jnp.maximum(raccs[i], v)
                    cmbuf[pl.ds(gl, L)] = cm
                    aibuf[pl.ds(gl, L)] = ai
                for i in range(RB):
                    racc[r0 + i] = raccs[i]
                return 0

            lax.fori_loop(0, G // RB, rb_body, 0)

            def pre_body(g, _):
                gl = g * L
                cm = cmbuf[pl.ds(gl, L)]
                ai = aibuf[pl.ds(gl, L)]
                neg = cm < 0.4
                pos = (cm > 0.5) & (cm < 0.8)
                a = jnp.where(neg, jnp.zeros((L,), jnp.int32),
                              jnp.full((L,), -1, jnp.int32))
                a = jnp.where(pos, ai + 1, a)
                prebuf[pl.ds(gl, L)] = a
                return 0
            lax.fori_loop(0, GPC, pre_body, 0)

            pltpu.sync_copy(cmbuf, maxov_hbm.at[pl.ds(c * W, W)])
            pltpu.sync_copy(prebuf, pre_hbm.at[pl.ds(c * W, W)])

    start(0, buf0, sem0)

    def outer(kk, _):
        k0 = 2 * kk
        start(k0 + 1, buf1, sem1)
        compute(k0, buf0, sem0)
        start(k0 + 2, buf0, sem0)
        compute(k0 + 1, buf1, sem1)
        return 0
    lax.fori_loop(0, KMAX // 2, outer, 0)

    pltpu.sync_copy(racc, part_hbm.at[w])


@functools.partial(
    pl.kernel,
    out_type=jax.ShapeDtypeStruct((N,), jnp.int32),
    mesh=_MESH,
    compiler_params=_PARAMS,
    scratch_types=[
        pltpu.VMEM((NW, G, L), jnp.float32),
        pltpu.VMEM((G, L), jnp.float32),
        pltpu.VMEM((W,), jnp.int32),
        pltpu.VMEM((W,), jnp.float32),
        pltpu.SMEM((G,), jnp.float32),
        pltpu.SMEM((G,), jnp.int32),
    ],
)
def _k2(ov_hbm, pre_hbm, part_hbm, asg_hbm,
        pall, gtb, outbuf, rowbuf, gts, rows):
    w = _widx()
    pltpu.sync_copy(part_hbm, pall)

    def gt_body(r, _):
        def w_body(w2, acc):
            return jnp.maximum(acc, pall[w2, r])
        acc = lax.fori_loop(0, NW, w_body, jnp.full((L,), -1.0, jnp.float32))
        s = jnp.max(acc)
        gtb[r] = jnp.zeros((L,), jnp.float32) + s
        gts[r] = s
        return 0
    lax.fori_loop(0, G, gt_body, 0)

    def cand_body(r, cnt):
        tie = jnp.max(pall[w, r]) == gts[r]

        @pl.when(tie)
        def _():
            rows[cnt] = r
        return jnp.where(tie, cnt + 1, cnt)
    ncand = lax.fori_loop(0, G, cand_body, jnp.int32(0))

    def chunk_body(k, _):
        c = k * NW + w

        @pl.when(c < NCH)
        def _():
            pltpu.sync_copy(pre_hbm.at[pl.ds(c * W, W)], outbuf)

            def row_body(i, _):
                r = rows[i]
                pltpu.sync_copy(ov_hbm.at[r, pl.ds(c * W, W)], rowbuf)
                rp1 = jnp.zeros((L,), jnp.int32) + (r + 1)

                def g_body(g, _):
                    gl = g * L
                    v = rowbuf[pl.ds(gl, L)]
                    m = v == gtb[r]
                    outbuf[pl.ds(gl, L)] = jnp.where(m, rp1,
                                                     outbuf[pl.ds(gl, L)])
                    return 0
                lax.fori_loop(0, GPC, g_body, 0)
                return 0
            lax.fori_loop(0, ncand, row_body, 0)

            pltpu.sync_copy(outbuf, asg_hbm.at[pl.ds(c * W, W)])
        return 0

    lax.fori_loop(0, KMAX, chunk_body, 0)


def kernel(overlaps):
    maxov, pre, part = _k1(overlaps)
    assigned = _k2(overlaps, pre, part)
    return assigned, maxov


# speedup vs baseline: 1.4051x; 1.4051x over previous
import functools

import jax
import jax.numpy as jnp
from jax import lax
from jax.experimental import pallas as pl
from jax.experimental.pallas import tpu as pltpu
from jax.experimental.pallas import tpu_sc as plsc

G = 128
N = 200000
L = 16
W = 400
GPC = W // L
GU = 5
NCH = N // W
NC = 2
NS = 16
NW = NC * NS
KMAX = (NCH + NW - 1) // NW
RB = 8
RCAP = 8

_MESH = plsc.VectorSubcoreMesh(core_axis_name="c", subcore_axis_name="s")
_PARAMS = pltpu.CompilerParams(use_tc_tiling_on_sc=False,
                               needs_layout_passes=False)


def _widx():
    return lax.axis_index("s") * NC + lax.axis_index("c")


def _splat_f(x):
    return jnp.zeros((L,), jnp.float32) + x


def _splat_i(x):
    return jnp.zeros((L,), jnp.int32) + x


@functools.partial(
    pl.kernel,
    out_type=[
        jax.ShapeDtypeStruct((N,), jnp.float32),
        jax.ShapeDtypeStruct((N,), jnp.int32),
        jax.ShapeDtypeStruct((NW, G, L), jnp.float32),
    ],
    mesh=_MESH,
    compiler_params=_PARAMS,
    scratch_types=[
        pltpu.VMEM((G, W), jnp.float32),
        pltpu.VMEM((G, W), jnp.float32),
        pltpu.VMEM((G, L), jnp.float32),
        pltpu.VMEM((KMAX, W), jnp.float32),
        pltpu.VMEM((KMAX, W), jnp.int32),
        pltpu.VMEM((W,), jnp.int32),
        pltpu.SemaphoreType.DMA,
        pltpu.SemaphoreType.DMA,
        pltpu.SemaphoreType.DMA,
    ],
)
def _k1(ov_hbm, maxov_hbm, pre_hbm, part_hbm,
        buf0, buf1, racc, cmall, preall, aibuf, sem0, sem1, semo):
    w = _widx()

    def init_racc(r, _):
        racc[r] = _splat_f(-1.0)
        return 0
    lax.fori_loop(0, G, init_racc, 0)

    def chunk_of(k):
        return k * NW + w

    def start(k, buf, sem):
        c = chunk_of(k)

        @pl.when(c < NCH)
        def _():
            pltpu.make_async_copy(
                ov_hbm.at[:, pl.ds(c * W, W)], buf, sem).start()

    def compute(k, buf, sem):
        c = chunk_of(k)

        @pl.when(c < NCH)
        def _():
            pltpu.make_async_copy(
                ov_hbm.at[:, pl.ds(c * W, W)], buf, sem).wait()

            def init_g(g, _):
                cmall[k, pl.ds(g * L, L)] = _splat_f(-1.0)
                aibuf[pl.ds(g * L, L)] = _splat_i(0)
                return 0
            lax.fori_loop(0, GPC, init_g, 0)

            zero = _splat_i(0)
            one = _splat_i(1)

            def rb_body(rb, _):
                r0 = rb * RB
                rvec = _splat_i(r0)
                raccs = [racc[r0 + i] for i in range(RB)]

                def g_body(g2, rs):
                    rs = list(rs)
                    for gu in range(GU):
                        gl = (g2 * GU + gu) * L
                        cm = cmall[k, pl.ds(gl, L)]
                        ai = aibuf[pl.ds(gl, L)]
                        vs = [buf[r0 + i, pl.ds(gl, L)] for i in range(RB)]
                        m01 = vs[1] > vs[0]
                        v01 = jnp.maximum(vs[0], vs[1])
                        i01 = jnp.where(m01, one, zero)
                        m23 = vs[3] > vs[2]
                        v23 = jnp.maximum(vs[2], vs[3])
                        i23 = jnp.where(m23, one, zero)
                        m45 = vs[5] > vs[4]
                        v45 = jnp.maximum(vs[4], vs[5])
                        i45 = jnp.where(m45, one, zero)
                        m67 = vs[7] > vs[6]
                        v67 = jnp.maximum(vs[6], vs[7])
                        i67 = jnp.where(m67, one, zero)
                        ma = v23 > v01
                        va = jnp.maximum(v01, v23)
                        ia = jnp.where(ma, i23 + 2, i01)
                        mb = v67 > v45
                        vb = jnp.maximum(v45, v67)
                        ib = jnp.where(mb, i67 + 2, i45)
                        mt = vb > va
                        vt = jnp.maximum(va, vb)
                        it = jnp.where(mt, ib + 4, ia)
                        mm = vt > cm
                        cmall[k, pl.ds(gl, L)] = jnp.maximum(cm, vt)
                        aibuf[pl.ds(gl, L)] = jnp.where(mm, it + rvec, ai)
                        for i in range(RB):
                            rs[i] = jnp.maximum(rs[i], vs[i])
                    return tuple(rs)

                fin = lax.fori_loop(0, GPC // GU, g_body, tuple(raccs))
                for i in range(RB):
                    racc[r0 + i] = fin[i]
                return 0

            lax.fori_loop(0, G // RB, rb_body, 0)

            def pre_body(g, _):
                gl = g * L
                cm = cmall[k, pl.ds(gl, L)]
                ai = aibuf[pl.ds(gl, L)]
                neg = cm < 0.4
                pos = (cm > 0.5) & (cm < 0.8)
                a = jnp.where(neg, zero, _splat_i(-1))
                a = jnp.where(pos, ai + 1, a)
                preall[k, pl.ds(gl, L)] = a
                return 0
            lax.fori_loop(0, GPC, pre_body, 0)

    start(0, buf0, sem0)

    def outer(kk, _):
        k0 = 2 * kk
        start(k0 + 1, buf1, sem1)
        compute(k0, buf0, sem0)
        start(k0 + 2, buf0, sem0)
        compute(k0 + 1, buf1, sem1)
        return 0
    lax.fori_loop(0, KMAX // 2, outer, 0)

    def out_start(k, _):
        c = chunk_of(k)

        @pl.when(c < NCH)
        def _():
            pltpu.make_async_copy(
                cmall.at[k], maxov_hbm.at[pl.ds(c * W, W)], semo).start()
            pltpu.make_async_copy(
                preall.at[k], pre_hbm.at[pl.ds(c * W, W)], semo).start()
        return 0
    lax.fori_loop(0, KMAX, out_start, 0)

    def out_wait(k, _):
        c = chunk_of(k)

        @pl.when(c < NCH)
        def _():
            pltpu.make_async_copy(
                cmall.at[k], maxov_hbm.at[pl.ds(c * W, W)], semo).wait()
            pltpu.make_async_copy(
                preall.at[k], pre_hbm.at[pl.ds(c * W, W)], semo).wait()
        return 0
    lax.fori_loop(0, KMAX, out_wait, 0)

    pltpu.sync_copy(racc, part_hbm.at[w])


@functools.partial(
    pl.kernel,
    out_type=jax.ShapeDtypeStruct((N,), jnp.int32),
    mesh=_MESH,
    compiler_params=_PARAMS,
    scratch_types=[
        pltpu.VMEM((8, G, L), jnp.float32),
        pltpu.VMEM((G, L), jnp.float32),
        pltpu.VMEM((G, L), jnp.float32),
        pltpu.VMEM((KMAX, W), jnp.int32),
        pltpu.VMEM((RCAP, KMAX, W), jnp.float32),
        pltpu.SMEM((G,), jnp.float32),
        pltpu.SMEM((G,), jnp.int32),
        pltpu.SemaphoreType.DMA,
        pltpu.SemaphoreType.DMA,
        pltpu.SemaphoreType.DMA,
    ],
)
def _k2(ov_hbm, pre_hbm, part_hbm, asg_hbm,
        slab, mypart, gtb, preall, rowall, gts, rows, semp, semr, semo):
    w = _widx()

    def chunk_of(k):
        return k * NW + w

    def pre_start(k, _):
        c = chunk_of(k)

        @pl.when(c < NCH)
        def _():
            pltpu.make_async_copy(
                pre_hbm.at[pl.ds(c * W, W)], preall.at[k], semp).start()
        return 0
    lax.fori_loop(0, KMAX, pre_start, 0)

    pltpu.sync_copy(part_hbm.at[w], mypart)

    def init_gtb(r, _):
        gtb[r] = _splat_f(-1.0)
        return 0
    lax.fori_loop(0, G, init_gtb, 0)

    def slab_body(wb, _):
        pltpu.sync_copy(part_hbm.at[pl.ds(wb * 8, 8)], slab)

        def r_body(r, _):
            vs = [slab[i, r] for i in range(8)]
            a = jnp.maximum(jnp.maximum(vs[0], vs[1]),
                            jnp.maximum(vs[2], vs[3]))
            b = jnp.maximum(jnp.maximum(vs[4], vs[5]),
                            jnp.maximum(vs[6], vs[7]))
            gtb[r] = jnp.maximum(gtb[r], jnp.maximum(a, b))
            return 0
        lax.fori_loop(0, G, r_body, 0)
        return 0
    lax.fori_loop(0, NW // 8, slab_body, 0)

    def fin_body(r, cnt):
        s = jnp.max(gtb[r])
        gtb[r] = _splat_f(s)
        gts[r] = s
        tie = jnp.max(mypart[r]) == s

        @pl.when(tie)
        def _():
            rows[cnt] = r
        return jnp.where(tie, cnt + 1, cnt)
    ncand = lax.fori_loop(0, G, fin_body, jnp.int32(0))

    def pre_wait(k, _):
        c = chunk_of(k)

        @pl.when(c < NCH)
        def _():
            pltpu.make_async_copy(
                pre_hbm.at[pl.ds(c * W, W)], preall.at[k], semp).wait()
        return 0
    lax.fori_loop(0, KMAX, pre_wait, 0)

    nbat = (ncand + RCAP - 1) // RCAP

    def bat_body(b, _):
        nb = jnp.minimum(ncand - b * RCAP, RCAP)

        def fire(i, _):
            r = rows[b * RCAP + i]

            def fk(k, _):
                c = chunk_of(k)

                @pl.when(c < NCH)
                def _():
                    pltpu.make_async_copy(
                        ov_hbm.at[r, pl.ds(c * W, W)], rowall.at[i, k],
                        semr).start()
                return 0
            lax.fori_loop(0, KMAX, fk, 0)
            return 0
        lax.fori_loop(0, nb, fire, 0)

        def drain(i, _):
            r = rows[b * RCAP + i]

            def fk(k, _):
                c = chunk_of(k)

                @pl.when(c < NCH)
                def _():
                    pltpu.make_async_copy(
                        ov_hbm.at[r, pl.ds(c * W, W)], rowall.at[i, k],
                        semr).wait()
                return 0
            lax.fori_loop(0, KMAX, fk, 0)
            return 0
        lax.fori_loop(0, nb, drain, 0)

        def patch_k(k, _):
            c = chunk_of(k)

            @pl.when(c < NCH)
            def _():
                def patch_i(i, _):
                    r = rows[b * RCAP + i]
                    rp1 = _splat_i(r + 1)

                    def patch_g(g, _):
                        gl = g * L
                        v = rowall[i, k, pl.ds(gl, L)]
                        m = v == gtb[r]
                        preall[k, pl.ds(gl, L)] = jnp.where(
                            m, rp1, preall[k, pl.ds(gl, L)])
                        return 0
                    lax.fori_loop(0, GPC, patch_g, 0)
                    return 0
                lax.fori_loop(0, nb, patch_i, 0)
            return 0
        lax.fori_loop(0, KMAX, patch_k, 0)
        return 0
    lax.fori_loop(0, nbat, bat_body, 0)

    def out_start(k, _):
        c = chunk_of(k)

        @pl.when(c < NCH)
        def _():
            pltpu.make_async_copy(
                preall.at[k], asg_hbm.at[pl.ds(c * W, W)], semo).start()
        return 0
    lax.fori_loop(0, KMAX, out_start, 0)

    def out_wait(k, _):
        c = chunk_of(k)

        @pl.when(c < NCH)
        def _():
            pltpu.make_async_copy(
                preall.at[k], asg_hbm.at[pl.ds(c * W, W)], semo).wait()
        return 0
    lax.fori_loop(0, KMAX, out_wait, 0)


def kernel(overlaps):
    maxov, pre, part = _k1(overlaps)
    assigned = _k2(overlaps, pre, part)
    return assigned, maxov


# speedup vs baseline: 1.9080x; 1.3580x over previous
import functools

import jax
import jax.numpy as jnp
from jax import lax
from jax.experimental import pallas as pl
from jax.experimental.pallas import tpu as pltpu
from jax.experimental.pallas import tpu_sc as plsc

G = 128
N = 200000
L = 16
W = 256
WT = N % W
NCH = N // W + 1
TAIL = NCH - 1
GPC = W // L
GPCT = WT // L
NC = 2
NS = 16
NW = NC * NS
KMAX = (NCH + NW - 1) // NW
RB = 8

_MESH = plsc.VectorSubcoreMesh(core_axis_name="c", subcore_axis_name="s")
_PARAMS = pltpu.CompilerParams(use_tc_tiling_on_sc=True,
                               needs_layout_passes=False)


def _widx():
    return lax.axis_index("s") * NC + lax.axis_index("c")


def _splat_f(x):
    return jnp.zeros((L,), jnp.float32) + x


def _splat_i(x):
    return jnp.zeros((L,), jnp.int32) + x


@functools.partial(
    pl.kernel,
    out_type=[
        jax.ShapeDtypeStruct((N,), jnp.float32),
        jax.ShapeDtypeStruct((N,), jnp.int32),
        jax.ShapeDtypeStruct((NW * G * L,), jnp.float32),
    ],
    mesh=_MESH,
    compiler_params=_PARAMS,
    scratch_types=[
        pltpu.VMEM((G, W), jnp.float32),
        pltpu.VMEM((G, W), jnp.float32),
        pltpu.VMEM((G, WT), jnp.float32),
        pltpu.VMEM((G * L,), jnp.float32),
        pltpu.VMEM((KMAX * W,), jnp.float32),
        pltpu.VMEM((KMAX * W,), jnp.int32),
        pltpu.VMEM((W,), jnp.int32),
        pltpu.SemaphoreType.DMA,
        pltpu.SemaphoreType.DMA,
        pltpu.SemaphoreType.DMA,
    ],
)
def _k1(ov_hbm, maxov_hbm, pre_hbm, part_hbm,
        buf0, buf1, buft, racc, cmall, preall, aibuf, sem0, sem1, semo):
    w = _widx()

    def init_racc(r, _):
        racc[pl.ds(r * L, L)] = _splat_f(-1.0)
        return 0
    lax.fori_loop(0, G, init_racc, 0)

    def chunk_of(k):
        return k * NW + w

    def start(k, buf, sem):
        c = chunk_of(k)

        @pl.when(c < TAIL)
        def _():
            pltpu.make_async_copy(
                ov_hbm.at[:, pl.ds(c * W, W)], buf, sem).start()

        @pl.when(c == TAIL)
        def _():
            pltpu.make_async_copy(
                ov_hbm.at[:, pl.ds(TAIL * W, WT)], buft, sem).start()

    def body(k, buf, width, gpc):
        zero = _splat_i(0)

        def init_g(g, _):
            cmall[pl.ds(k * W + g * L, L)] = _splat_f(-1.0)
            aibuf[pl.ds(g * L, L)] = zero
            return 0
        lax.fori_loop(0, gpc, init_g, 0)

        one = _splat_i(1)

        def rb_body(rb, _):
            r0 = rb * RB
            rvec = _splat_i(r0)
            raccs = [racc[pl.ds((r0 + i) * L, L)] for i in range(RB)]

            def g_body(g2, rs):
                rs = list(rs)
                for gu in range(4):
                    g = g2 * 4 + gu
                    gl = g * L
                    cm = cmall[pl.ds(k * W + gl, L)]
                    ai = aibuf[pl.ds(gl, L)]
                    vs = [buf[r0 + i, pl.ds(gl, L)] for i in range(RB)]
                    m01 = vs[1] > vs[0]
                    v01 = jnp.maximum(vs[0], vs[1])
                    i01 = jnp.where(m01, one, zero)
                    m23 = vs[3] > vs[2]
                    v23 = jnp.maximum(vs[2], vs[3])
                    i23 = jnp.where(m23, one, zero)
                    m45 = vs[5] > vs[4]
                    v45 = jnp.maximum(vs[4], vs[5])
                    i45 = jnp.where(m45, one, zero)
                    m67 = vs[7] > vs[6]
                    v67 = jnp.maximum(vs[6], vs[7])
                    i67 = jnp.where(m67, one, zero)
                    ma = v23 > v01
                    va = jnp.maximum(v01, v23)
                    ia = jnp.where(ma, i23 + 2, i01)
                    mb = v67 > v45
                    vb = jnp.maximum(v45, v67)
                    ib = jnp.where(mb, i67 + 2, i45)
                    mt = vb > va
                    vt = jnp.maximum(va, vb)
                    it = jnp.where(mt, ib + 4, ia)
                    mm = vt > cm
                    cmall[pl.ds(k * W + gl, L)] = jnp.maximum(cm, vt)
                    aibuf[pl.ds(gl, L)] = jnp.where(mm, it + rvec, ai)
                    for i in range(RB):
                        rs[i] = jnp.maximum(rs[i], vs[i])
                return tuple(rs)

            fin = lax.fori_loop(0, gpc // 4, g_body, tuple(raccs))
            for i in range(RB):
                racc[pl.ds((r0 + i) * L, L)] = fin[i]
            return 0

        lax.fori_loop(0, G // RB, rb_body, 0)

        def pre_body(g, _):
            gl = g * L
            cm = cmall[pl.ds(k * W + gl, L)]
            ai = aibuf[pl.ds(gl, L)]
            neg = cm < 0.4
            pos = (cm > 0.5) & (cm < 0.8)
            a = jnp.where(neg, zero, _splat_i(-1))
            a = jnp.where(pos, ai + 1, a)
            preall[pl.ds(k * W + gl, L)] = a
            return 0
        lax.fori_loop(0, gpc, pre_body, 0)

    def compute(k, buf, sem):
        c = chunk_of(k)

        @pl.when(c < TAIL)
        def _():
            pltpu.make_async_copy(
                ov_hbm.at[:, pl.ds(c * W, W)], buf, sem).wait()
            body(k, buf, W, GPC)

        @pl.when(c == TAIL)
        def _():
            pltpu.make_async_copy(
                ov_hbm.at[:, pl.ds(TAIL * W, WT)], buft, sem).wait()
            body(k, buft, WT, GPCT)

    start(0, buf0, sem0)

    def outer(kk, _):
        k0 = 2 * kk
        start(k0 + 1, buf1, sem1)
        compute(k0, buf0, sem0)
        start(k0 + 2, buf0, sem0)
        compute(k0 + 1, buf1, sem1)
        return 0
    lax.fori_loop(0, KMAX // 2, outer, 0)
    compute(KMAX - 1, buf0, sem0)

    def flush(k, do):
        c = chunk_of(k)

        @pl.when(c < TAIL)
        def _():
            do(pltpu.make_async_copy(
                cmall.at[pl.ds(k * W, W)],
                maxov_hbm.at[pl.ds(c * W, W)], semo))
            do(pltpu.make_async_copy(
                preall.at[pl.ds(k * W, W)],
                pre_hbm.at[pl.ds(c * W, W)], semo))

        @pl.when(c == TAIL)
        def _():
            do(pltpu.make_async_copy(
                cmall.at[pl.ds(k * W, WT)],
                maxov_hbm.at[pl.ds(TAIL * W, WT)], semo))
            do(pltpu.make_async_copy(
                preall.at[pl.ds(k * W, WT)],
                pre_hbm.at[pl.ds(TAIL * W, WT)], semo))

    def out_start(k, _):
        flush(k, lambda cp: cp.start())
        return 0
    lax.fori_loop(0, KMAX, out_start, 0)

    def out_wait(k, _):
        flush(k, lambda cp: cp.wait())
        return 0
    lax.fori_loop(0, KMAX, out_wait, 0)

    pltpu.sync_copy(racc, part_hbm.at[pl.ds(w * G * L, G * L)])


@functools.partial(
    pl.kernel,
    out_type=jax.ShapeDtypeStruct((N,), jnp.int32),
    mesh=_MESH,
    compiler_params=_PARAMS,
    scratch_types=[
        pltpu.VMEM((8 * G * L,), jnp.float32),
        pltpu.VMEM((G * L,), jnp.float32),
        pltpu.VMEM((G * L,), jnp.float32),
        pltpu.VMEM((KMAX * W,), jnp.int32),
        pltpu.VMEM((KMAX, 8, W), jnp.float32),
        pltpu.VMEM((8, WT), jnp.float32),
        pltpu.SMEM((G,), jnp.float32),
        pltpu.SMEM((G,), jnp.int32),
        pltpu.SemaphoreType.DMA,
        pltpu.SemaphoreType.DMA,
        pltpu.SemaphoreType.DMA,
    ],
)
def _k2(ov_hbm, pre_hbm, part_hbm, asg_hbm,
        slab, mypart, gtb, preall, rowall, bandt, gts, rows,
        semp, semr, semo):
    w = _widx()

    def chunk_of(k):
        return k * NW + w

    def pre_flush(k, do, src, dst):
        c = chunk_of(k)

        @pl.when(c < TAIL)
        def _():
            do(pltpu.make_async_copy(
                src.at[pl.ds(c * W, W)], dst.at[pl.ds(k * W, W)], semp))

        @pl.when(c == TAIL)
        def _():
            do(pltpu.make_async_copy(
                src.at[pl.ds(TAIL * W, WT)], dst.at[pl.ds(k * W, WT)], semp))

    def pre_start(k, _):
        pre_flush(k, lambda cp: cp.start(), pre_hbm, preall)
        return 0
    lax.fori_loop(0, KMAX, pre_start, 0)

    pltpu.sync_copy(part_hbm.at[pl.ds(w * G * L, G * L)], mypart)

    def init_gtb(r, _):
        gtb[pl.ds(r * L, L)] = _splat_f(-1.0)
        return 0
    lax.fori_loop(0, G, init_gtb, 0)

    def slab_body(wb, _):
        pltpu.sync_copy(part_hbm.at[pl.ds(wb * 8 * G * L, 8 * G * L)], slab)

        def r_body(r, _):
            vs = [slab[pl.ds((i * G + r) * L, L)] for i in range(8)]
            a = jnp.maximum(jnp.maximum(vs[0], vs[1]),
                            jnp.maximum(vs[2], vs[3]))
            b = jnp.maximum(jnp.maximum(vs[4], vs[5]),
                            jnp.maximum(vs[6], vs[7]))
            gl = pl.ds(r * L, L)
            gtb[gl] = jnp.maximum(gtb[gl], jnp.maximum(a, b))
            return 0
        lax.fori_loop(0, G, r_body, 0)
        return 0
    lax.fori_loop(0, NW // 8, slab_body, 0)

    def fin_body(r, cnt):
        s = jnp.max(gtb[pl.ds(r * L, L)])
        gtb[pl.ds(r * L, L)] = _splat_f(s)
        gts[r] = s
        tie = jnp.max(mypart[pl.ds(r * L, L)]) == s

        @pl.when(tie)
        def _():
            rows[cnt] = r
        return jnp.where(tie, cnt + 1, cnt)
    ncand = lax.fori_loop(0, G, fin_body, jnp.int32(0))

    def pre_wait(k, _):
        pre_flush(k, lambda cp: cp.wait(), pre_hbm, preall)
        return 0
    lax.fori_loop(0, KMAX, pre_wait, 0)

    def band(k, r8, do):
        c = chunk_of(k)

        @pl.when(c < TAIL)
        def _():
            do(pltpu.make_async_copy(
                ov_hbm.at[pl.ds(r8, 8), pl.ds(c * W, W)],
                rowall.at[k], semr))

        @pl.when(c == TAIL)
        def _():
            do(pltpu.make_async_copy(
                ov_hbm.at[pl.ds(r8, 8), pl.ds(TAIL * W, WT)],
                bandt, semr))

    def bat_body(b, _):
        r = rows[b]
        r8 = pl.multiple_of((r // 8) * 8, 8)
        ri = r - r8
        rp1 = _splat_i(r + 1)
        gv = gtb[pl.ds(r * L, L)]

        def fire(k, _):
            band(k, r8, lambda cp: cp.start())
            return 0
        lax.fori_loop(0, KMAX, fire, 0)

        def drain(k, _):
            band(k, r8, lambda cp: cp.wait())
            return 0
        lax.fori_loop(0, KMAX, drain, 0)

        def patch_k(k, _):
            c = chunk_of(k)

            def patch(gpc, src):
                def patch_g(g, _):
                    gl = g * L
                    v = src(gl)
                    m = v == gv
                    sl = pl.ds(k * W + gl, L)
                    preall[sl] = jnp.where(m, rp1, preall[sl])
                    return 0
                lax.fori_loop(0, gpc, patch_g, 0)

            @pl.when(c < TAIL)
            def _():
                patch(GPC, lambda gl: rowall[k, ri, pl.ds(gl, L)])

            @pl.when(c == TAIL)
            def _():
                patch(GPCT, lambda gl: bandt[ri, pl.ds(gl, L)])
            return 0
        lax.fori_loop(0, KMAX, patch_k, 0)
        return 0
    lax.fori_loop(0, ncand, bat_body, 0)

    def asg_flush(k, do):
        c = chunk_of(k)

        @pl.when(c < TAIL)
        def _():
            do(pltpu.make_async_copy(
                preall.at[pl.ds(k * W, W)], asg_hbm.at[pl.ds(c * W, W)],
                semo))

        @pl.when(c == TAIL)
        def _():
            do(pltpu.make_async_copy(
                preall.at[pl.ds(k * W, WT)],
                asg_hbm.at[pl.ds(TAIL * W, WT)], semo))

    def asg_start(k, _):
        asg_flush(k, lambda cp: cp.start())
        return 0
    lax.fori_loop(0, KMAX, asg_start, 0)

    def asg_wait(k, _):
        asg_flush(k, lambda cp: cp.wait())
        return 0
    lax.fori_loop(0, KMAX, asg_wait, 0)


def kernel(overlaps):
    maxov, pre, part = _k1(overlaps)
    assigned = _k2(overlaps, pre, part)
    return assigned, maxov
